# baseline (speedup 1.0000x reference)
import functools

import jax
import jax.numpy as jnp
import numpy as np
from jax import lax
from jax.experimental import pallas as pl
from jax.experimental.pallas import tpu as pltpu
from jax.experimental.pallas import tpu_sc as plsc

D_MODEL = 36
N_ROWS = 32768
NW = 32
ROWS_W = N_ROWS // NW
CHUNK_W = ROWS_W * D_MODEL
GROUPS = ROWS_W // 16

_mesh = plsc.VectorSubcoreMesh(core_axis_name="c", subcore_axis_name="s")


def _wid():
    return lax.axis_index("s") * 2 + lax.axis_index("c")


def _rne(x):
    u = plsc.bitcast(x, jnp.uint32)
    u = u + jnp.uint32(0x7FFF) + ((u >> jnp.uint32(16)) & jnp.uint32(1))
    u = u & jnp.uint32(0xFFFF0000)
    return plsc.bitcast(u, jnp.float32)


@functools.partial(
    pl.kernel,
    out_type=[
        jax.ShapeDtypeStruct((NW, 16), jnp.float32),
        jax.ShapeDtypeStruct((NW, 16), jnp.int32),
    ],
    mesh=_mesh,
    compiler_params=pltpu.CompilerParams(needs_layout_passes=False),
    scratch_types=[
        pltpu.VMEM((CHUNK_W,), jnp.float32),
        pltpu.VMEM((64,), jnp.float32),
        pltpu.VMEM((128,), jnp.float32),
        pltpu.VMEM((128,), jnp.float32),
        pltpu.VMEM((16,), jnp.float32),
        pltpu.VMEM((16,), jnp.int32),
        pltpu.SemaphoreType.DMA,
    ],
)
def _stage1(qe_h, wq_h, wk_h, m_h, osc_h, oid_h,
            m_v, qe_v, wq_v, wk_v, bs_v, bi_v, sem):
    wid = _wid()
    base = wid * CHUNK_W
    cp = pltpu.async_copy(m_h.at[pl.ds(base, CHUNK_W)], m_v, sem)
    pltpu.sync_copy(qe_h, qe_v)
    pltpu.sync_copy(wq_h, wq_v)
    pltpu.sync_copy(wk_h, wk_v)

    q0 = jnp.float32(0.0)
    q1 = jnp.float32(0.0)
    for t in range(3):
        qe_t = _rne(qe_v[pl.ds(16 * t, 16)])
        p0 = _rne(wq_v[pl.ds(16 * t, 16)]) * qe_t
        p1 = _rne(wq_v[pl.ds(64 + 16 * t, 16)]) * qe_t
        for l in range(16):
            q0 = q0 + p0[l]
            q1 = q1 + p1[l]
    q0b = _rne(jnp.full((16,), q0, jnp.float32))[0]
    q1b = _rne(jnp.full((16,), q1, jnp.float32))[0]
    wkb = [_rne(wk_v[pl.ds(16 * t, 16)]) for t in range(3)]
    wkb2 = [_rne(wk_v[pl.ds(64 + 16 * t, 16)]) for t in range(3)]
    wk0 = [wkb[d // 16][d % 16] for d in range(D_MODEL)]
    wk1 = [wkb2[d // 16][d % 16] for d in range(D_MODEL)]

    iota = lax.iota(jnp.int32, 16)
    cp.wait()

    def group(g, carry):
        bv, bi = carry
        rowv = g * 16 + iota
        idx0 = rowv * D_MODEL
        a0 = jnp.zeros((16,), jnp.float32)
        a1 = jnp.zeros((16,), jnp.float32)
        for d in range(D_MODEL):
            gv = _rne(plsc.load_gather(m_v, [idx0 + d]))
            a0 = a0 + gv * wk0[d]
            a1 = a1 + gv * wk1[d]
        acc = _rne(a0) * q0b + _rne(a1) * q1b
        upd = acc > bv
        bv = jnp.where(upd, acc, bv)
        bi = jnp.where(upd, wid * ROWS_W + rowv, bi)
        return bv, bi

    bv0 = jnp.full((16,), -jnp.inf, jnp.float32)
    bi0 = jnp.zeros((16,), jnp.int32)
    bv, bi = lax.fori_loop(0, GROUPS, group, (bv0, bi0))

    bs_v[...] = bv
    bi_v[...] = bi
    pltpu.sync_copy(bs_v, osc_h.at[wid])
    pltpu.sync_copy(bi_v, oid_h.at[wid])


@functools.partial(
    pl.kernel,
    out_type=[
        jax.ShapeDtypeStruct((16,), jnp.float32),
        jax.ShapeDtypeStruct((16,), jnp.float32),
        jax.ShapeDtypeStruct((16,), jnp.int32),
    ],
    mesh=_mesh,
    compiler_params=pltpu.CompilerParams(needs_layout_passes=False),
    scratch_types=[
        pltpu.VMEM((NW, 16), jnp.float32),
        pltpu.VMEM((NW, 16), jnp.int32),
        pltpu.VMEM((64,), jnp.float32),
        pltpu.VMEM((16,), jnp.float32),
        pltpu.VMEM((16,), jnp.float32),
        pltpu.VMEM((16,), jnp.float32),
        pltpu.VMEM((16,), jnp.float32),
        pltpu.VMEM((16,), jnp.float32),
        pltpu.VMEM((16,), jnp.int32),
        pltpu.SemaphoreType.DMA,
    ],
)
def _stage2(osc_h, oid_h, m_h, wv_h, oval_h, osco_h, oidx_h,
            sc_v, id_v, wv_v, r0_v, r1_v, r2_v, ov_v, os_v, oi_v, sem):
    wid = _wid()

    @pl.when(wid == 0)
    def _():
        pltpu.sync_copy(osc_h, sc_v)
        pltpu.sync_copy(oid_h, id_v)
        pltpu.sync_copy(wv_h, wv_v)
        iota = lax.iota(jnp.int32, 16)

        bv = sc_v[0]
        bi = id_v[0]
        for w in range(1, NW):
            s = sc_v[w]
            i = id_v[w]
            upd = (s > bv) | ((s == bv) & (i < bi))
            bv = jnp.where(upd, s, bv)
            bi = jnp.where(upd, i, bi)
        mx = jnp.float32(-jnp.inf)
        bidx = jnp.int32(0)
        for j in range(16):
            s = bv[j]
            i = bi[j]
            upd = (s > mx) | ((s == mx) & (i < bidx))
            mx = jnp.where(upd, s, mx)
            bidx = jnp.where(upd, i, bidx)

        eb = bidx * D_MODEL
        pltpu.async_copy(m_h.at[eb + iota], r0_v, sem).wait()
        pltpu.async_copy(m_h.at[eb + 16 + iota], r1_v, sem).wait()
        pltpu.async_copy(m_h.at[eb + 20 + iota], r2_v, sem).wait()

        p0 = _rne(r0_v[...]) * _rne(wv_v[pl.ds(0, 16)])
        p1 = _rne(r1_v[...]) * _rne(wv_v[pl.ds(16, 16)])
        p2 = _rne(r2_v[...]) * _rne(wv_v[pl.ds(20, 16)])
        val = jnp.float32(0.0)
        for d in range(16):
            val = val + p0[d]
        for d in range(16):
            val = val + p1[d]
        for d in range(12, 16):
            val = val + p2[d]

        ov_v[...] = jnp.full((16,), val, jnp.float32)
        os_v[...] = jnp.full((16,), mx, jnp.float32)
        oi_v[...] = jnp.full((16,), bidx, jnp.int32)
        pltpu.sync_copy(ov_v, oval_h)
        pltpu.sync_copy(os_v, osco_h)
        pltpu.sync_copy(oi_v, oidx_h)


@jax.jit
def kernel(query_emb, memory_embs, W_Q, W_K, W_V):
    f32 = jnp.float32
    qe = jnp.zeros((64,), f32).at[:D_MODEL].set(query_emb)
    wq = jnp.zeros((2, 64), f32).at[:, :D_MODEL].set(W_Q).reshape(128)
    wk = jnp.zeros((2, 64), f32).at[:, :D_MODEL].set(W_K).reshape(128)
    wv = jnp.zeros((64,), f32).at[:D_MODEL].set(W_V[0])
    m_flat = memory_embs.reshape(N_ROWS * D_MODEL)

    osc, oid = _stage1(qe, wq, wk, m_flat)
    val, sco, idx = _stage2(osc, oid, m_flat, wv)
    return (val[0:1], sco[0], idx[0])


# speedup vs baseline: 1.1873x; 1.1873x over previous
import functools

import jax
import jax.numpy as jnp
import numpy as np
from jax import lax
from jax.experimental import pallas as pl
from jax.experimental.pallas import tpu as pltpu
from jax.experimental.pallas import tpu_sc as plsc

D_MODEL = 36
N_ROWS = 32768
NW = 32
ROWS_W = N_ROWS // NW
CHUNK_W = ROWS_W * D_MODEL
GROUPS = ROWS_W // 16

_mesh = plsc.VectorSubcoreMesh(core_axis_name="c", subcore_axis_name="s")


def _wid():
    return lax.axis_index("s") * 2 + lax.axis_index("c")


def _rne(x):
    u = plsc.bitcast(x, jnp.uint32)
    u = u + jnp.uint32(0x7FFF) + ((u >> jnp.uint32(16)) & jnp.uint32(1))
    u = u & jnp.uint32(0xFFFF0000)
    return plsc.bitcast(u, jnp.float32)


@functools.partial(
    pl.kernel,
    out_type=[
        jax.ShapeDtypeStruct((NW, 16), jnp.float32),
        jax.ShapeDtypeStruct((NW, 16), jnp.int32),
    ],
    mesh=_mesh,
    compiler_params=pltpu.CompilerParams(needs_layout_passes=False),
    scratch_types=[
        pltpu.VMEM((CHUNK_W,), jnp.float32),
        pltpu.VMEM((64,), jnp.float32),
        pltpu.VMEM((128,), jnp.float32),
        pltpu.VMEM((128,), jnp.float32),
        pltpu.VMEM((16,), jnp.float32),
        pltpu.VMEM((16,), jnp.int32),
        pltpu.SemaphoreType.DMA,
    ],
)
def _stage1(qe_h, wq_h, wk_h, m_h, osc_h, oid_h,
            m_v, qe_v, wq_v, wk_v, bs_v, bi_v, sem):
    wid = _wid()
    base = wid * CHUNK_W
    cp = pltpu.async_copy(m_h.at[pl.ds(base, CHUNK_W)], m_v, sem)
    pltpu.sync_copy(qe_h, qe_v)
    pltpu.sync_copy(wq_h, wq_v)
    pltpu.sync_copy(wk_h, wk_v)

    q0 = jnp.float32(0.0)
    q1 = jnp.float32(0.0)
    for t in range(3):
        qe_t = _rne(qe_v[pl.ds(16 * t, 16)])
        p0 = _rne(wq_v[pl.ds(16 * t, 16)]) * qe_t
        p1 = _rne(wq_v[pl.ds(64 + 16 * t, 16)]) * qe_t
        for l in range(16):
            q0 = q0 + p0[l]
            q1 = q1 + p1[l]
    q0b = _rne(jnp.full((16,), q0, jnp.float32))[0]
    q1b = _rne(jnp.full((16,), q1, jnp.float32))[0]
    wkb = [_rne(wk_v[pl.ds(16 * t, 16)]) for t in range(3)]
    wkb2 = [_rne(wk_v[pl.ds(64 + 16 * t, 16)]) for t in range(3)]
    wk0 = [wkb[d // 16][d % 16] for d in range(D_MODEL)]
    wk1 = [wkb2[d // 16][d % 16] for d in range(D_MODEL)]

    iota = lax.iota(jnp.int32, 16)
    cp.wait()

    def rpass(t, _):
        for u in range(8):
            off = t * 128 + u * 16
            m_v[pl.ds(off, 16)] = _rne(m_v[pl.ds(off, 16)])
        return 0
    lax.fori_loop(0, CHUNK_W // 128, rpass, 0)

    def group(g, carry):
        bv, bi = carry
        rowv = g * 16 + iota
        idx0 = rowv * D_MODEL
        a0 = jnp.zeros((16,), jnp.float32)
        a1 = jnp.zeros((16,), jnp.float32)
        for d in range(D_MODEL):
            gv = plsc.load_gather(m_v, [idx0 + d])
            a0 = a0 + gv * wk0[d]
            a1 = a1 + gv * wk1[d]
        acc = _rne(a0) * q0b + _rne(a1) * q1b
        upd = acc > bv
        bv = jnp.where(upd, acc, bv)
        bi = jnp.where(upd, wid * ROWS_W + rowv, bi)
        return bv, bi

    bv0 = jnp.full((16,), -jnp.inf, jnp.float32)
    bi0 = jnp.zeros((16,), jnp.int32)
    bv, bi = lax.fori_loop(0, GROUPS, group, (bv0, bi0))

    bs_v[...] = bv
    bi_v[...] = bi
    pltpu.sync_copy(bs_v, osc_h.at[wid])
    pltpu.sync_copy(bi_v, oid_h.at[wid])


def _merge_body(osc_ref, oid_ref, wv_ref, m_ref,
                val_ref, sco_ref, idx_ref, row_v, sem):
    s = osc_ref[...]
    i = oid_ref[...]
    mx = jnp.max(s)
    best = jnp.min(jnp.where(s == mx, i, N_ROWS))

    cp = pltpu.make_async_copy(m_ref.at[pl.ds(best, 1), :], row_v, sem)
    cp.start()
    cp.wait()

    rb = row_v[...].astype(jnp.bfloat16).astype(jnp.float32)
    wb = wv_ref[...].astype(jnp.bfloat16).astype(jnp.float32)
    val_ref[...] = jnp.sum(rb * wb, axis=1, keepdims=True)
    sco_ref[...] = jnp.full((1, 1), mx, jnp.float32)
    idx_ref[...] = jnp.full((1, 1), best, jnp.int32)


@jax.jit
def kernel(query_emb, memory_embs, W_Q, W_K, W_V):
    f32 = jnp.float32
    qe = jnp.zeros((64,), f32).at[:D_MODEL].set(query_emb)
    wq = jnp.zeros((2, 64), f32).at[:, :D_MODEL].set(W_Q).reshape(128)
    wk = jnp.zeros((2, 64), f32).at[:, :D_MODEL].set(W_K).reshape(128)
    m_flat = memory_embs.reshape(N_ROWS * D_MODEL)

    osc, oid = _stage1(qe, wq, wk, m_flat)

    val, sco, idx = pl.pallas_call(
        _merge_body,
        in_specs=[
            pl.BlockSpec((NW, 16), lambda: (0, 0)),
            pl.BlockSpec((NW, 16), lambda: (0, 0)),
            pl.BlockSpec((1, D_MODEL), lambda: (0, 0)),
            pl.BlockSpec(memory_space=pl.ANY),
        ],
        out_specs=[
            pl.BlockSpec((1, 1), lambda: (0, 0)),
            pl.BlockSpec((1, 1), lambda: (0, 0)),
            pl.BlockSpec((1, 1), lambda: (0, 0)),
        ],
        out_shape=[
            jax.ShapeDtypeStruct((1, 1), jnp.float32),
            jax.ShapeDtypeStruct((1, 1), jnp.float32),
            jax.ShapeDtypeStruct((1, 1), jnp.int32),
        ],
        scratch_shapes=[
            pltpu.VMEM((1, D_MODEL), jnp.float32),
            pltpu.SemaphoreType.DMA,
        ],
    )(osc, oid, W_V, memory_embs)
    return (val.reshape(1), sco[0, 0], idx[0, 0])
